# Initial kernel scaffold; baseline (speedup 1.0000x reference)
import jax
import jax.numpy as jnp
from jax.experimental import pallas as pl


def kernel(token_ids, weight):
    raise NotImplementedError("write your pallas kernel here")


# speedup vs baseline: 4.5394x; 4.5394x over previous
import functools

import jax
import jax.numpy as jnp
from jax import lax
from jax.experimental import pallas as pl
from jax.experimental.pallas import tpu as pltpu
from jax.experimental.pallas import tpu_sc as plsc

D = 64
TOTAL = 4096 * 50
NC, NS = 2, 16
NW = NC * NS
PER_W = TOTAL // NW
CHUNK = 128
NCHUNK = PER_W // CHUNK
NBUF = 2

_mesh = plsc.VectorSubcoreMesh(core_axis_name="c", subcore_axis_name="s")


@functools.partial(
    pl.kernel,
    mesh=_mesh,
    out_type=jax.ShapeDtypeStruct((TOTAL, D), jnp.float32),
    scratch_types=[
        pltpu.VMEM((NCHUNK, CHUNK), jnp.int32),
        pltpu.VMEM((NBUF, CHUNK, D), jnp.float32),
        pltpu.SemaphoreType.DMA,
        pltpu.SemaphoreType.DMA,
        pltpu.SemaphoreType.DMA,
        pltpu.SemaphoreType.DMA,
    ],
    compiler_params=pltpu.CompilerParams(use_tc_tiling_on_sc=False),
)
def _emb_lookup(idx_hbm, table_hbm, out_hbm, idx_v, rows_v,
                gsem0, gsem1, wsem0, wsem1):
    wid = lax.axis_index("s") * NC + lax.axis_index("c")
    base = wid * PER_W

    pltpu.sync_copy(idx_hbm.at[wid], idx_v)

    gsems = (gsem0, gsem1)
    wsems = (wsem0, wsem1)

    def gather(c, b):
        pltpu.async_copy(table_hbm.at[idx_v.at[c]], rows_v.at[b], gsems[b])

    def writeback(c, b):
        pltpu.async_copy(rows_v.at[b],
                         out_hbm.at[pl.ds(base + c * CHUNK, CHUNK)],
                         wsems[b])

    for b in range(NBUF):
        gather(b, b)

    def body(c0, _):
        for b in range(NBUF):
            c = c0 + b
            pltpu.make_async_copy(table_hbm.at[idx_v.at[c]],
                                  rows_v.at[b], gsems[b]).wait()
            writeback(c, b)
            nxt = c + NBUF

            @pl.when(nxt < NCHUNK)
            def _():
                pltpu.make_async_copy(rows_v.at[b],
                                      out_hbm.at[pl.ds(base + c * CHUNK, CHUNK)],
                                      wsems[b]).wait()
                gather(nxt, b)
        return 0

    lax.fori_loop(0, NCHUNK // NBUF, lambda i, x: body(i * NBUF, x), 0,
                  unroll=False)

    for b in range(NBUF):
        c_last = NCHUNK - NBUF + b
        pltpu.make_async_copy(rows_v.at[b],
                              out_hbm.at[pl.ds(base + c_last * CHUNK, CHUNK)],
                              wsems[b]).wait()


def kernel(token_ids, weight):
    idx = token_ids.reshape(NW, NCHUNK, CHUNK).astype(jnp.int32)
    out = _emb_lookup(idx, weight)
    return out.reshape(token_ids.shape + (D,))


# speedup vs baseline: 4.6719x; 1.0292x over previous
import functools

import jax
import jax.numpy as jnp
from jax import lax
from jax.experimental import pallas as pl
from jax.experimental.pallas import tpu as pltpu
from jax.experimental.pallas import tpu_sc as plsc

D = 64
TOTAL = 4096 * 50
NC, NS = 2, 16
NW = NC * NS
PER_W = TOTAL // NW
SEG = 128
SUB = 2
CHUNK = SEG * SUB
NCHUNK = PER_W // CHUNK
NSEG = PER_W // SEG
NBUF = 5
LOOK = 2

_mesh = plsc.VectorSubcoreMesh(core_axis_name="c", subcore_axis_name="s")


@functools.partial(
    pl.kernel,
    mesh=_mesh,
    out_type=jax.ShapeDtypeStruct((TOTAL, D), jnp.float32),
    scratch_types=[
        pltpu.VMEM((NSEG, SEG), jnp.int32),
        pltpu.VMEM((NBUF, CHUNK, D), jnp.float32),
        [pltpu.SemaphoreType.DMA] * NBUF,
        [pltpu.SemaphoreType.DMA] * NBUF,
    ],
    compiler_params=pltpu.CompilerParams(use_tc_tiling_on_sc=False),
)
def _emb_lookup(idx_hbm, table_hbm, out_hbm, idx_v, rows_v, gsems, wsems):
    wid = lax.axis_index("s") * NC + lax.axis_index("c")
    base = wid * PER_W

    pltpu.sync_copy(idx_hbm.at[wid], idx_v)

    def fire_gathers(c, b):
        for j in range(SUB):
            pltpu.async_copy(table_hbm.at[idx_v.at[c * SUB + j]],
                             rows_v.at[b, pl.ds(j * SEG, SEG)], gsems[b])

    def wait_gathers(c, b):
        for j in range(SUB):
            pltpu.make_async_copy(table_hbm.at[idx_v.at[c * SUB + j]],
                                  rows_v.at[b, pl.ds(j * SEG, SEG)],
                                  gsems[b]).wait()

    def fire_writeback(c, b):
        pltpu.async_copy(rows_v.at[b],
                         out_hbm.at[pl.ds(base + c * CHUNK, CHUNK)], wsems[b])

    def wait_writeback(c, b):
        pltpu.make_async_copy(rows_v.at[b],
                              out_hbm.at[pl.ds(base + c * CHUNK, CHUNK)],
                              wsems[b]).wait()

    for b in range(LOOK):
        fire_gathers(b, b)

    def step(c, b):
        wait_gathers(c, b)
        fire_writeback(c, b)
        n = c + LOOK
        bn = (b + LOOK) % NBUF

        @pl.when(n < NCHUNK)
        def _():
            @pl.when(n >= NBUF)
            def _():
                wait_writeback(n - NBUF, bn)

            fire_gathers(n, bn)
        return 0

    lax.fori_loop(
        0, NCHUNK // NBUF,
        lambda i, x: [step(i * NBUF + b, b) for b in range(NBUF)][-1],
        0, unroll=False)

    for m in range(NCHUNK - NBUF, NCHUNK):
        wait_writeback(m, m % NBUF)


def kernel(token_ids, weight):
    idx = token_ids.reshape(NW, NSEG, SEG).astype(jnp.int32)
    out = _emb_lookup(idx, weight)
    return out.reshape(token_ids.shape + (D,))
